# baseline (speedup 1.0000x reference)
import functools

import jax
import jax.numpy as jnp
from jax import lax
from jax.experimental import pallas as pl
from jax.experimental.pallas import tpu as pltpu
from jax.experimental.pallas import tpu_sc as plsc


def kernel(x, tok_emb, pos_emb):
    B, T = x.shape
    V, D = tok_emb.shape
    L = 16

    info = plsc.get_sparse_core_info()
    NC, NS = info.num_cores, info.num_subcores
    NW = NC * NS
    N = B * T
    per_w = N // NW
    C = 32
    nch = per_w // C

    mesh = plsc.VectorSubcoreMesh(core_axis_name="c", subcore_axis_name="s")

    @functools.partial(
        pl.kernel,
        mesh=mesh,
        out_type=jax.ShapeDtypeStruct((N, D), jnp.float32),
        scratch_types=[
            pltpu.VMEM((nch, C), jnp.int32),
            pltpu.VMEM((C, D), jnp.float32),
            pltpu.VMEM((C, D), jnp.float32),
            pltpu.SemaphoreType.DMA,
        ],
    )
    def emb_kernel(x_hbm, tok_hbm, pos_hbm, out_hbm, idx_v, rows_v, pos_v, sem):
        wid = lax.axis_index("s") * NC + lax.axis_index("c")
        base = wid * per_w
        t0 = base % T

        pltpu.sync_copy(x_hbm.at[wid], idx_v)

        def chunk_body(ch, _):
            row0 = base + ch * C
            gat = pltpu.async_copy(tok_hbm.at[idx_v.at[ch]], rows_v, sem)
            pltpu.sync_copy(pos_hbm.at[pl.ds(t0 + ch * C, C)], pos_v)
            gat.wait()

            def add_body(i, _):
                r = i // (D // L)
                col = (i % (D // L)) * L
                rows_v[r, pl.ds(col, L)] = (
                    rows_v[r, pl.ds(col, L)] + pos_v[r, pl.ds(col, L)]
                )
                return 0

            lax.fori_loop(0, C * (D // L), add_body, 0)
            pltpu.sync_copy(rows_v, out_hbm.at[pl.ds(row0, C)])
            return 0

        lax.fori_loop(0, nch, chunk_body, 0)

    x3 = x.reshape(NW, nch, C)
    out = emb_kernel(x3, tok_emb, pos_emb)
    return out.reshape(B, T, D)


# speedup vs baseline: 3.0492x; 3.0492x over previous
import functools

import jax
import jax.numpy as jnp
from jax import lax
from jax.experimental import pallas as pl
from jax.experimental.pallas import tpu as pltpu
from jax.experimental.pallas import tpu_sc as plsc


def kernel(x, tok_emb, pos_emb):
    B, T = x.shape
    V, D = tok_emb.shape
    L = 16

    info = plsc.get_sparse_core_info()
    NC, NS = info.num_cores, info.num_subcores
    NW = NC * NS
    t_span = T // NW
    C = 16
    nch = t_span // C
    G = nch * B
    NBUF = 3
    VPR = D // L

    mesh = plsc.VectorSubcoreMesh(core_axis_name="c", subcore_axis_name="s")

    @functools.partial(
        pl.kernel,
        mesh=mesh,
        out_type=jax.ShapeDtypeStruct((B * T, D), jnp.float32),
        scratch_types=[
            pltpu.VMEM((nch, B, C), jnp.int32),
            pltpu.VMEM((NBUF, C, D), jnp.float32),
            pltpu.VMEM((C, D), jnp.float32),
            pltpu.SemaphoreType.DMA,
            pltpu.SemaphoreType.DMA,
            pltpu.SemaphoreType.DMA,
        ],
    )
    def emb_kernel(x_hbm, tok_hbm, pos_hbm, out_hbm, idx_v, rows_v, pos_v,
                   sem_g, sem_o, sem_p):
        wid = lax.axis_index("s") * NC + lax.axis_index("c")
        t0 = wid * t_span

        pltpu.sync_copy(x_hbm.at[wid], idx_v)
        pos_cp = pltpu.async_copy(pos_hbm.at[pl.ds(t0, C)], pos_v, sem_p)

        gathers = [None] * G
        outs = [None] * G
        gathers[0] = pltpu.async_copy(
            tok_hbm.at[idx_v.at[0, 0]], rows_v.at[0], sem_g)

        for g in range(G):
            ch, b = divmod(g, B)
            slot = g % NBUF
            if g + 1 < G:
                ch2, b2 = divmod(g + 1, B)
                if g >= NBUF - 1:
                    outs[g + 1 - NBUF].wait()
                gathers[g + 1] = pltpu.async_copy(
                    tok_hbm.at[idx_v.at[ch2, b2]],
                    rows_v.at[(g + 1) % NBUF], sem_g)
            if b == 0:
                pos_cp.wait()
            gathers[g].wait()

            @plsc.parallel_loop(0, C * VPR, unroll=8)
            def add_body(i):
                r = i // VPR
                col = (i % VPR) * L
                rows_v[slot, r, pl.ds(col, L)] = (
                    rows_v[slot, r, pl.ds(col, L)] + pos_v[r, pl.ds(col, L)]
                )

            if b == B - 1 and ch + 1 < nch:
                pos_cp = pltpu.async_copy(
                    pos_hbm.at[pl.ds(t0 + (ch + 1) * C, C)], pos_v, sem_p)
            row0 = b * T + t0 + ch * C
            outs[g] = pltpu.async_copy(
                rows_v.at[slot], out_hbm.at[pl.ds(row0, C)], sem_o)

        for g in range(max(0, G - NBUF), G):
            outs[g].wait()

    x3 = x.reshape(B, NW, nch, C).transpose(1, 2, 0, 3)
    out = emb_kernel(x3, tok_emb, pos_emb)
    return out.reshape(B, T, D)


# speedup vs baseline: 3.5168x; 1.1534x over previous
import functools

import jax
import jax.numpy as jnp
from jax import lax
from jax.experimental import pallas as pl
from jax.experimental.pallas import tpu as pltpu
from jax.experimental.pallas import tpu_sc as plsc


def kernel(x, tok_emb, pos_emb):
    B, T = x.shape
    V, D = tok_emb.shape
    L = 16

    info = plsc.get_sparse_core_info()
    NC, NS = info.num_cores, info.num_subcores
    NW = NC * NS
    t_span = T // NW
    C = 16
    nch = t_span // C
    G = nch * B
    NBUF = 4
    VPR = D // L

    mesh = plsc.VectorSubcoreMesh(core_axis_name="c", subcore_axis_name="s")

    @functools.partial(
        pl.kernel,
        mesh=mesh,
        out_type=jax.ShapeDtypeStruct((B * T, D), jnp.float32),
        scratch_types=[
            pltpu.VMEM((nch, B, C), jnp.int32),
            pltpu.VMEM((NBUF, C, D), jnp.float32),
            pltpu.VMEM((2, C, D), jnp.float32),
            pltpu.SemaphoreType.DMA,
            pltpu.SemaphoreType.DMA,
            pltpu.SemaphoreType.DMA,
        ],
    )
    def emb_kernel(x_hbm, tok_hbm, pos_hbm, out_hbm, idx_v, rows_v, pos_v,
                   sem_g, sem_o, sem_p):
        wid = lax.axis_index("s") * NC + lax.axis_index("c")
        t0 = wid * t_span

        pltpu.sync_copy(x_hbm.at[wid], idx_v)
        pos_cp = [
            pltpu.async_copy(pos_hbm.at[pl.ds(t0 + c * C, C)], pos_v.at[c],
                             sem_p)
            for c in range(2)
        ]

        gathers = [None] * G
        outs = [None] * G
        for g in range(NBUF - 1):
            ch, b = divmod(g, B)
            gathers[g] = pltpu.async_copy(
                tok_hbm.at[idx_v.at[ch, b]], rows_v.at[g % NBUF], sem_g)

        for g in range(G):
            ch, b = divmod(g, B)
            slot = g % NBUF
            gathers[g].wait()
            if b == 0:
                pos_cp[ch % 2].wait()

            @plsc.parallel_loop(0, C * VPR, unroll=8)
            def add_body(i):
                r = i // VPR
                col = (i % VPR) * L
                rows_v[slot, r, pl.ds(col, L)] = (
                    rows_v[slot, r, pl.ds(col, L)]
                    + pos_v[ch % 2, r, pl.ds(col, L)]
                )

            if b == B - 1 and ch + 2 < nch:
                pos_cp[ch % 2] = pltpu.async_copy(
                    pos_hbm.at[pl.ds(t0 + (ch + 2) * C, C)],
                    pos_v.at[ch % 2], sem_p)
            row0 = b * T + t0 + ch * C
            outs[g] = pltpu.async_copy(
                rows_v.at[slot], out_hbm.at[pl.ds(row0, C)], sem_o)

            ng = g + NBUF - 1
            if ng < G:
                if g >= 1:
                    outs[g - 1].wait()
                ch2, b2 = divmod(ng, B)
                gathers[ng] = pltpu.async_copy(
                    tok_hbm.at[idx_v.at[ch2, b2]], rows_v.at[ng % NBUF],
                    sem_g)

        for g in range(max(0, G - NBUF), G):
            outs[g].wait()

    x3 = x.reshape(B, NW, nch, C).transpose(1, 2, 0, 3)
    out = emb_kernel(x3, tok_emb, pos_emb)
    return out.reshape(B, T, D)
